# baseline (speedup 1.0000x reference)
import jax
import jax.numpy as jnp
from jax import lax
from jax.experimental import pallas as pl
from jax.experimental.pallas import tpu as pltpu


def _gcn_kernel(seq_ref, w_ref, adj_ref, bias_ref, a_ref, out_ref, sf_ref):
    @pl.when(pl.program_id(0) == 0)
    def _():
        sf_ref[...] = lax.dot_general(
            seq_ref[...], w_ref[...], (((1,), (1,)), ((), ())),
            preferred_element_type=jnp.float32)

    acc = jnp.dot(adj_ref[...], sf_ref[...], preferred_element_type=jnp.float32)
    o = acc + bias_ref[...]
    out_ref[...] = jnp.where(o >= 0, o, a_ref[0, 0] * o)


def kernel(seq, adj, W, bias, prelu_a):
    N, d_in = seq.shape
    d_out = W.shape[0]
    bm = 232
    bias2 = bias.reshape(1, d_out)
    a2 = jnp.reshape(prelu_a, (1, 1))
    out = pl.pallas_call(
        _gcn_kernel,
        grid=(pl.cdiv(N, bm),),
        in_specs=[
            pl.BlockSpec((N, d_in), lambda i: (0, 0)),
            pl.BlockSpec((d_out, d_in), lambda i: (0, 0)),
            pl.BlockSpec((bm, N), lambda i: (i, 0)),
            pl.BlockSpec((1, d_out), lambda i: (0, 0)),
            pl.BlockSpec((1, 1), lambda i: (0, 0)),
        ],
        out_specs=pl.BlockSpec((bm, d_out), lambda i: (i, 0)),
        out_shape=jax.ShapeDtypeStruct((N, d_out), jnp.float32),
        scratch_shapes=[pltpu.VMEM((N, d_out), jnp.float32)],
        compiler_params=pltpu.CompilerParams(
            dimension_semantics=("arbitrary",),
            vmem_limit_bytes=110 * 1024 * 1024,
        ),
    )(seq, W, adj, bias2, a2)
    return out[None]


# speedup vs baseline: 1.0102x; 1.0102x over previous
import jax
import jax.numpy as jnp
from jax import lax
from jax.experimental import pallas as pl
from jax.experimental.pallas import tpu as pltpu


def _gcn_kernel(seq_ref, w_ref, adj_ref, bias_ref, a_ref, out_ref, sf_ref):
    @pl.when(pl.program_id(0) == 0)
    def _():
        sf_ref[...] = lax.dot_general(
            seq_ref[...], w_ref[...], (((1,), (1,)), ((), ())),
            preferred_element_type=jnp.float32)

    acc = jnp.dot(adj_ref[...], sf_ref[...], preferred_element_type=jnp.float32)
    o = acc + bias_ref[...]
    out_ref[...] = jnp.where(o >= 0, o, a_ref[0, 0] * o)


def kernel(seq, adj, W, bias, prelu_a):
    N, d_in = seq.shape
    d_out = W.shape[0]
    bm = 240
    bias2 = bias.reshape(1, d_out)
    a2 = jnp.reshape(prelu_a, (1, 1))
    out = pl.pallas_call(
        _gcn_kernel,
        grid=(pl.cdiv(N, bm),),
        in_specs=[
            pl.BlockSpec((N, d_in), lambda i: (0, 0)),
            pl.BlockSpec((d_out, d_in), lambda i: (0, 0)),
            pl.BlockSpec((bm, N), lambda i: (i, 0)),
            pl.BlockSpec((1, d_out), lambda i: (0, 0)),
            pl.BlockSpec((1, 1), lambda i: (0, 0)),
        ],
        out_specs=pl.BlockSpec((bm, d_out), lambda i: (i, 0)),
        out_shape=jax.ShapeDtypeStruct((N, d_out), jnp.float32),
        scratch_shapes=[pltpu.VMEM((N, d_out), jnp.float32)],
        compiler_params=pltpu.CompilerParams(
            dimension_semantics=("arbitrary",),
            vmem_limit_bytes=110 * 1024 * 1024,
        ),
    )(seq, W, adj, bias2, a2)
    return out[None]
